# baseline (speedup 1.0000x reference)
import functools

import jax
import jax.numpy as jnp
from jax import lax
from jax.experimental import pallas as pl
from jax.experimental.pallas import tpu as pltpu
from jax.experimental.pallas import tpu_sc as plsc

V = 1000000
B = 16384
D = 64
NC = 2
NS = 16
NW = NC * NS
BPW = B // NW
CHUNK = 128
NCH = BPW // CHUNK
WD = 4
NWAVE = D // WD
LANES = 16
VPC = CHUNK // LANES


def _sc_embed_lookup(X, tab_flat, shared_flat):
    mesh = plsc.VectorSubcoreMesh(core_axis_name="c", subcore_axis_name="s")

    @functools.partial(
        pl.kernel,
        mesh=mesh,
        out_type=jax.ShapeDtypeStruct((D, B), jnp.float32),
        compiler_params=pltpu.CompilerParams(use_tc_tiling_on_sc=False),
        scratch_types=[
            pltpu.VMEM((D * NCH, CHUNK), jnp.int32),
            pltpu.VMEM((D, BPW), jnp.float32),
            pltpu.VMEM((D, LANES), jnp.float32),
            pltpu.SemaphoreType.DMA,
            pltpu.SemaphoreType.DMA,
        ],
    )
    def body(x_hbm, tab_hbm, sh_hbm, out_hbm, ib, rows, sh_v, gsem, osem):
        wid = lax.axis_index("s") * NC + lax.axis_index("c")
        base = wid * BPW

        pltpu.sync_copy(sh_hbm, sh_v)
        for j in range(NCH):
            pltpu.sync_copy(
                x_hbm.at[pl.ds(base + j * CHUNK, CHUNK)], ib.at[j]
            )

        def build_dim(d):
            off = d * jnp.int32(V)
            for j in range(NCH):
                for k in range(VPC):
                    sl = pl.ds(k * LANES, LANES)
                    ib[NCH * d + j, sl] = ib[j, sl] + off

        def fire_wave(w):
            for t in range(WD * NCH):
                dd = WD * w + t // NCH
                jj = t % NCH
                pltpu.async_copy(
                    tab_hbm.at[ib.at[NCH * dd + jj]],
                    rows.at[dd, pl.ds(jj * CHUNK, CHUNK)],
                    gsem,
                )

        def drain_wave(w):
            pltpu.make_async_copy(
                out_hbm.at[pl.ds(WD * w, WD), pl.ds(base, BPW)],
                rows.at[pl.ds(WD * w, WD)],
                gsem,
            ).wait()

        def add_wave(w):
            for t in range(WD):
                dd = WD * w + t
                sv = sh_v[dd, :]
                for v in range(BPW // LANES):
                    plsc.addupdate(rows.at[dd, pl.ds(v * LANES, LANES)], sv)

        def out_wave(w):
            pltpu.async_copy(
                rows.at[pl.ds(WD * w, WD)],
                out_hbm.at[pl.ds(WD * w, WD), pl.ds(base, BPW)],
                osem,
            )

        for d in range(1, WD):
            build_dim(jnp.int32(d))
        fire_wave(jnp.int32(0))

        def wave_body(g, carry):
            for t in range(WD):
                build_dim(WD * (g + 1) + t)
            fire_wave(g + 1)
            drain_wave(g)
            add_wave(g)
            out_wave(g)
            return carry

        lax.fori_loop(0, NWAVE - 1, wave_body, 0)

        last = jnp.int32(NWAVE - 1)
        drain_wave(last)
        add_wave(last)
        out_wave(last)
        pltpu.make_async_copy(
            out_hbm.at[pl.ds(0, D), pl.ds(base, BPW)], rows, osem
        ).wait()

    return body(X, tab_flat, shared_flat)


def kernel(X, embed_table, shared_embed):
    tab_flat = embed_table.T.reshape(D * V)
    sh_bcast = jnp.broadcast_to(shared_embed.reshape(D, 1), (D, LANES))
    out_t = _sc_embed_lookup(X, tab_flat, sh_bcast)
    return out_t.T
